# baseline (speedup 1.0000x reference)
import functools

import jax
import jax.numpy as jnp
from jax import lax
from jax.experimental import pallas as pl
from jax.experimental.pallas import tpu as pltpu
from jax.experimental.pallas import tpu_sc as plsc

N_NODES = 10000
N_EDGES = 160000
IN_CH = 256
EMB = 512
N_GRAPHS = 1000
N_SG = 50000
N_PAIRS = 4096

NC, NS, L = 2, 16, 16
NROWS = 10240
DUMMY_NODE = 10008
GPAD = 1024
DUMMY_G = 1000

EPS = 80 * 128
EPD = 40 * 128
SGP = 25 * 128

_f32 = jnp.float32
_i32 = jnp.int32

_MESH = plsc.VectorSubcoreMesh(core_axis_name="c", subcore_axis_name="s",
                               num_cores=NC, num_subcores=NS)


def _deg_body(dst4d, z1n, ones1, degp, acc, dstv, ones1v):
    c = lax.axis_index("c")
    s = lax.axis_index("s")
    pltpu.sync_copy(z1n.at[pl.ds(s * 640, 640)], acc.at[pl.ds(s * 640, 640)])
    pltpu.sync_copy(ones1, ones1v)
    pltpu.sync_copy(dst4d.at[c, s], dstv)
    plsc.subcore_barrier()

    def step(j, _):
        for k in range(128 // L):
            iv = dstv[j, pl.ds(k * L, L)]
            pltpu.sync_copy(ones1v, acc.at[iv], add=True)
        return _

    lax.fori_loop(0, EPD // 128, step, None)
    plsc.subcore_barrier()
    pltpu.sync_copy(acc.at[pl.ds(s * 640, 640)],
                    degp.at[c, pl.ds(s * 640, 640)])


_deg_sc = functools.partial(
    pl.kernel, _deg_body,
    out_type=jax.ShapeDtypeStruct((NC, NROWS), _f32),
    mesh=_MESH,
    scratch_types=[
        pltpu.VMEM_SHARED((NROWS,), _f32),
        pltpu.VMEM((EPD // 128, 128), _i32),
        pltpu.VMEM((L,), _f32),
    ],
)()


def _agg_body(hsc, src3d, dst3d, out, acc, srcv, dstv, rows):
    c = lax.axis_index("c")
    s = lax.axis_index("s")
    pltpu.sync_copy(src3d.at[s], srcv)
    pltpu.sync_copy(dst3d.at[s], dstv)
    for j2 in range(2):
        ch = c + 2 * j2
        pltpu.sync_copy(hsc.at[ch, pl.ds(s * 640, 640)],
                        acc.at[pl.ds(s * 640, 640)])
        plsc.subcore_barrier()

        def step(j, _):
            pltpu.sync_copy(hsc.at[ch].at[srcv.at[j]], rows)
            for k in range(128 // L):
                iv = dstv[j, pl.ds(k * L, L)]
                pltpu.sync_copy(rows.at[pl.ds(k * L, L)], acc.at[iv],
                                add=True)
            return _

        lax.fori_loop(0, EPS // 128, step, None)
        plsc.subcore_barrier()
        pltpu.sync_copy(acc.at[pl.ds(s * 640, 640)],
                        out.at[ch, pl.ds(s * 640, 640)])
        plsc.subcore_barrier()


_agg_sc = functools.partial(
    pl.kernel, _agg_body,
    out_type=jax.ShapeDtypeStruct((4, NROWS, 128), _f32),
    mesh=_MESH,
    scratch_types=[
        pltpu.VMEM_SHARED((NROWS, 128), _f32),
        pltpu.VMEM((EPS // 128, 128), _i32),
        pltpu.VMEM((EPS // 128, 128), _i32),
        pltpu.VMEM((128, 128), _f32),
    ],
)()


def _pool_body(h2c, node3d, avg3d, idsall, z2, z1g, ones1, sums4, cnt,
               presf, acc, cacc, pacc, nv, av, idpv, rows, ones1v):
    c = lax.axis_index("c")
    s = lax.axis_index("s")
    pltpu.sync_copy(z2.at[pl.ds(s * 128, 128)], acc.at[pl.ds(s * 128, 128)])

    @pl.when(jnp.logical_and(c == 0, s == 0))
    def _():
        pltpu.sync_copy(z1g, cacc)
        pltpu.sync_copy(z1g, pacc)

    pltpu.sync_copy(ones1, ones1v)
    pltpu.sync_copy(node3d.at[s], nv)
    pltpu.sync_copy(avg3d.at[s], av)
    pltpu.sync_copy(idsall.at[s], idpv)
    plsc.subcore_barrier()

    @pl.when(c == 0)
    def _():
        for j in range(4):
            for k in range(128 // L):
                iv = idpv[j, pl.ds(k * L, L)]
                pltpu.sync_copy(ones1v, pacc.at[iv])

    for j2 in range(2):
        ch = c + 2 * j2

        def step(j, _):
            pltpu.sync_copy(h2c.at[ch].at[nv.at[j]], rows)
            for k in range(128 // L):
                iv = av[j, pl.ds(k * L, L)] + GPAD * j2
                pltpu.sync_copy(rows.at[pl.ds(k * L, L)], acc.at[iv],
                                add=True)
            return _

        lax.fori_loop(0, SGP // 128, step, None)

    @pl.when(c == 0)
    def _():
        def cstep(j, _):
            for k in range(128 // L):
                iv = av[j, pl.ds(k * L, L)]
                pltpu.sync_copy(ones1v, cacc.at[iv], add=True)
            return _
        lax.fori_loop(0, SGP // 128, cstep, None)

    plsc.subcore_barrier()
    for j2 in range(2):
        pltpu.sync_copy(acc.at[pl.ds(GPAD * j2 + s * 64, 64)],
                        sums4.at[c + 2 * j2, pl.ds(s * 64, 64)])

    @pl.when(jnp.logical_and(c == 0, s == 0))
    def _():
        pltpu.sync_copy(cacc, cnt)
        pltpu.sync_copy(pacc, presf)


_pool_sc = functools.partial(
    pl.kernel, _pool_body,
    out_type=(jax.ShapeDtypeStruct((4, GPAD, 128), _f32),
              jax.ShapeDtypeStruct((GPAD,), _f32),
              jax.ShapeDtypeStruct((GPAD,), _f32)),
    mesh=_MESH,
    scratch_types=[
        pltpu.VMEM_SHARED((2 * GPAD, 128), _f32),
        pltpu.VMEM_SHARED((GPAD,), _f32),
        pltpu.VMEM_SHARED((GPAD,), _f32),
        pltpu.VMEM((SGP // 128, 128), _i32),
        pltpu.VMEM((SGP // 128, 128), _i32),
        pltpu.VMEM((4, 128), _i32),
        pltpu.VMEM((128, 128), _f32),
        pltpu.VMEM((L,), _f32),
    ],
)()


def _pairs_body(from2d, to2d, rankshbm, sums4, cnthbm, ftrows, cfo, cto,
                fidv, tidv, rtmp, ctmp, cfv, rlist, rows):
    c = lax.axis_index("c")
    s = lax.axis_index("s")
    w = c * NS + s
    pltpu.sync_copy(from2d.at[w], fidv)
    pltpu.sync_copy(to2d.at[w], tidv)

    for b, (idv, c_out) in enumerate(((fidv, cfo), (tidv, cto))):
        for k in range(128 // L):
            ids16 = idv[0, pl.ds(k * L, L)]
            pltpu.sync_copy(rankshbm.at[ids16], rtmp)
            r16 = rtmp[...] - 1
            rlist[pl.ds(k * L, L)] = r16
            pltpu.sync_copy(cnthbm.at[r16], ctmp)
            cfv[pl.ds(k * L, L)] = jnp.maximum(ctmp[...], 1.0)
        pltpu.sync_copy(cfv, c_out.at[pl.ds(128 * w, 128)])
        for h in range(4):
            for k in range(128 // L):
                iv = rlist[pl.ds(k * L, L)]
                pltpu.sync_copy(sums4.at[h].at[iv],
                                rows.at[pl.ds(k * L, L)])
            pltpu.sync_copy(rows, ftrows.at[b, h, pl.ds(128 * w, 128)])


_pairs_sc = functools.partial(
    pl.kernel, _pairs_body,
    out_type=(jax.ShapeDtypeStruct((2, 4, N_PAIRS, 128), _f32),
              jax.ShapeDtypeStruct((N_PAIRS,), _f32),
              jax.ShapeDtypeStruct((N_PAIRS,), _f32)),
    mesh=_MESH,
    scratch_types=[
        pltpu.VMEM((1, 128), _i32),
        pltpu.VMEM((1, 128), _i32),
        pltpu.VMEM((L,), _i32),
        pltpu.VMEM((L,), _f32),
        pltpu.VMEM((128,), _f32),
        pltpu.VMEM((128,), _i32),
        pltpu.VMEM((128, 128), _f32),
    ],
)()


def _rank_body(p_ref, r_ref):
    p = p_ref[...]
    row = lax.broadcasted_iota(_i32, (GPAD, GPAD), 0)
    col = lax.broadcasted_iota(_i32, (GPAD, GPAD), 1)
    m = jnp.where(col <= row, 1.0, 0.0)
    ranks = lax.dot_general(m, p, (((1,), (0,)), ((), ())),
                            precision=lax.Precision.HIGHEST,
                            preferred_element_type=_f32)
    r_ref[...] = jnp.round(ranks).astype(_i32)


def _rank_tc(presf):
    return pl.pallas_call(
        _rank_body,
        out_shape=jax.ShapeDtypeStruct((GPAD, 1), _i32),
    )(presf)


_BM = 1024


def _m1_body(x_ref, degp_ref, w_ref, hs_ref, dis_ref):
    dp = degp_ref[...]
    deg = 1.0 + dp[:, 0] + dp[:, 1]
    dis = lax.rsqrt(deg)
    h = jnp.dot(x_ref[...], w_ref[...], preferred_element_type=_f32)
    hs = h * dis[:, None]
    for c in range(4):
        hs_ref[c] = hs[:, 128 * c:128 * (c + 1)]
    dis_ref[...] = dis[:, None]


def _m1_tc(x, degp, W1):
    return pl.pallas_call(
        _m1_body,
        grid=(NROWS // _BM,),
        in_specs=[
            pl.BlockSpec((_BM, IN_CH), lambda i: (i, 0)),
            pl.BlockSpec((_BM, 2), lambda i: (i, 0)),
            pl.BlockSpec((IN_CH, EMB), lambda i: (0, 0)),
        ],
        out_specs=[
            pl.BlockSpec((4, _BM, 128), lambda i: (0, i, 0)),
            pl.BlockSpec((_BM, 1), lambda i: (i, 0)),
        ],
        out_shape=[
            jax.ShapeDtypeStruct((4, NROWS, 128), _f32),
            jax.ShapeDtypeStruct((NROWS, 1), _f32),
        ],
    )(x, degp, W1)


def _m2_body(agg_ref, dis_ref, b_ref, w_ref, hs_ref):
    a = jnp.concatenate([agg_ref[c] for c in range(4)], axis=-1)
    dis = dis_ref[...]
    h1 = jnp.maximum(a * dis + b_ref[...][0, :], 0.0)
    hs = jnp.dot(h1, w_ref[...], preferred_element_type=_f32) * dis
    for c in range(4):
        hs_ref[c] = hs[:, 128 * c:128 * (c + 1)]


def _m2_tc(aggc, dis, b1r, W2):
    return pl.pallas_call(
        _m2_body,
        grid=(NROWS // _BM,),
        in_specs=[
            pl.BlockSpec((4, _BM, 128), lambda i: (0, i, 0)),
            pl.BlockSpec((_BM, 1), lambda i: (i, 0)),
            pl.BlockSpec((1, EMB), lambda i: (0, 0)),
            pl.BlockSpec((EMB, EMB), lambda i: (0, 0)),
        ],
        out_specs=pl.BlockSpec((4, _BM, 128), lambda i: (0, i, 0)),
        out_shape=jax.ShapeDtypeStruct((4, NROWS, 128), _f32),
    )(aggc, dis, b1r, W2)


def _elem_body(agg_ref, dis_ref, b_ref, h2_ref):
    a = jnp.concatenate([agg_ref[c] for c in range(4)], axis=-1)
    h2 = jnp.maximum(a * dis_ref[...] + b_ref[...][0, :], 0.0)
    for c in range(4):
        h2_ref[c] = h2[:, 128 * c:128 * (c + 1)]


def _elem_tc(aggc, dis, b2r):
    return pl.pallas_call(
        _elem_body,
        grid=(NROWS // _BM,),
        in_specs=[
            pl.BlockSpec((4, _BM, 128), lambda i: (0, i, 0)),
            pl.BlockSpec((_BM, 1), lambda i: (i, 0)),
            pl.BlockSpec((1, EMB), lambda i: (0, 0)),
        ],
        out_specs=pl.BlockSpec((4, _BM, 128), lambda i: (0, i, 0)),
        out_shape=jax.ShapeDtypeStruct((4, NROWS, 128), _f32),
    )(aggc, dis, b2r)


def _dot_body(f_ref, t_ref, cf_ref, ct_ref, o_ref):
    f = f_ref[0]
    t = t_ref[0]
    sums = jnp.sum(f * t, axis=(0, 2))
    o_ref[...] = (sums / (cf_ref[...][:, 0] * ct_ref[...][:, 0]))[:, None]


def _dot_tc(frows, trows, cf2, ct2):
    bm = 1024
    return pl.pallas_call(
        _dot_body,
        grid=(N_PAIRS // bm,),
        in_specs=[
            pl.BlockSpec((1, 4, bm, 128), lambda i: (0, 0, i, 0)),
            pl.BlockSpec((1, 4, bm, 128), lambda i: (1, 0, i, 0)),
            pl.BlockSpec((bm, 1), lambda i: (i, 0)),
            pl.BlockSpec((bm, 1), lambda i: (i, 0)),
        ],
        out_specs=pl.BlockSpec((bm, 1), lambda i: (i, 0)),
        out_shape=jax.ShapeDtypeStruct((N_PAIRS, 1), _f32),
    )(frows, trows, cf2, ct2)


def kernel(x, drug_drug_batch, edge_cell_lines, sg_edge_index, sg_nodes,
           sg_avging_idx, W1, b1, W2, b2):
    del edge_cell_lines
    src = sg_edge_index[0].astype(_i32)
    dst = sg_edge_index[1].astype(_i32)

    pad_a = NS * EPS - N_EDGES
    src3d = jnp.concatenate([src, jnp.zeros((pad_a,), _i32)]
                            ).reshape(NS, EPS // 128, 128)
    dst3d = jnp.concatenate([dst, jnp.full((pad_a,), DUMMY_NODE, _i32)]
                            ).reshape(NS, EPS // 128, 128)
    pad_d = NC * NS * EPD - N_EDGES
    dst4d = jnp.concatenate([dst, jnp.full((pad_d,), DUMMY_NODE, _i32)]
                            ).reshape(NC, NS, EPD // 128, 128)
    pad_p = NS * SGP - N_SG
    node3d = jnp.concatenate([sg_nodes.astype(_i32),
                              jnp.zeros((pad_p,), _i32)]
                             ).reshape(NS, SGP // 128, 128)
    avg3d = jnp.concatenate([sg_avging_idx.astype(_i32),
                             jnp.full((pad_p,), DUMMY_G, _i32)]
                            ).reshape(NS, SGP // 128, 128)
    ddb = drug_drug_batch.astype(_i32)
    from2d = ddb[0].reshape(NC * NS, 1, 128)
    to2d = ddb[1].reshape(NC * NS, 1, 128)
    idsall = ddb.reshape(NS, 4, 128)

    z1n = jnp.zeros((NROWS,), _f32)
    z2 = jnp.zeros((2 * GPAD, 128), _f32)
    z1g = jnp.zeros((GPAD,), _f32)
    ones1f = jnp.ones((L,), _f32)

    degp = _deg_sc(dst4d, z1n, ones1f)
    xp = jnp.concatenate([x, jnp.zeros((NROWS - N_NODES, IN_CH), _f32)])
    hs1c, dis = _m1_tc(xp, degp.T, W1)
    agg1c = _agg_sc(hs1c, src3d, dst3d)
    hs2c = _m2_tc(agg1c, dis, b1.reshape(1, EMB), W2)
    agg2c = _agg_sc(hs2c, src3d, dst3d)
    h2c = _elem_tc(agg2c, dis, b2.reshape(1, EMB))
    sums4, cnt, presf = _pool_sc(h2c, node3d, avg3d, idsall, z2, z1g,
                                 ones1f)
    ranks = _rank_tc(presf.reshape(GPAD, 1)).reshape(GPAD)
    ftrows, cf, ct = _pairs_sc(from2d, to2d, ranks, sums4, cnt)
    out = _dot_tc(ftrows, ftrows, cf.reshape(N_PAIRS, 1),
                  ct.reshape(N_PAIRS, 1))
    return out.reshape(N_PAIRS)


# speedup vs baseline: 1.8180x; 1.8180x over previous
import functools

import jax
import jax.numpy as jnp
from jax import lax
from jax.experimental import pallas as pl
from jax.experimental.pallas import tpu as pltpu
from jax.experimental.pallas import tpu_sc as plsc

N_NODES = 10000
N_EDGES = 160000
IN_CH = 256
EMB = 512
N_GRAPHS = 1000
N_SG = 50000
N_PAIRS = 4096

NC, NS, L = 2, 16, 16
NROWS = 10240
DUMMY_NODE = 10008
GPAD = 1024
DUMMY_G = 1000

EPS = 79 * 128
EPD = 40 * 128
SGP = 25 * 128

_f32 = jnp.float32
_i32 = jnp.int32

_MESH = plsc.VectorSubcoreMesh(core_axis_name="c", subcore_axis_name="s",
                               num_cores=NC, num_subcores=NS)


def _deg_body(dst4d, z1n, ones1, degp, acc, dstv, ones1v):
    c = lax.axis_index("c")
    s = lax.axis_index("s")
    pltpu.sync_copy(z1n.at[pl.ds(s * 640, 640)], acc.at[pl.ds(s * 640, 640)])
    pltpu.sync_copy(ones1, ones1v)
    pltpu.sync_copy(dst4d.at[c, s], dstv)
    plsc.subcore_barrier()

    def step(j, _):
        for k in range(128 // L):
            iv = dstv[j, pl.ds(k * L, L)]
            pltpu.sync_copy(ones1v, acc.at[iv], add=True)
        return _

    lax.fori_loop(0, EPD // 128, step, None)
    plsc.subcore_barrier()
    pltpu.sync_copy(acc.at[pl.ds(s * 640, 640)],
                    degp.at[c, pl.ds(s * 640, 640)])


_deg_sc = functools.partial(
    pl.kernel, _deg_body,
    out_type=jax.ShapeDtypeStruct((NC, NROWS), _f32),
    mesh=_MESH,
    scratch_types=[
        pltpu.VMEM_SHARED((NROWS,), _f32),
        pltpu.VMEM((EPD // 128, 128), _i32),
        pltpu.VMEM((L,), _f32),
    ],
)()


def _agg_body(hsc, src3d, dst3d, out, acc, srcv, dstv, rows):
    c = lax.axis_index("c")
    s = lax.axis_index("s")
    pltpu.sync_copy(src3d.at[s], srcv)
    pltpu.sync_copy(dst3d.at[s], dstv)
    for j2 in range(2):
        ch = c + 2 * j2
        pltpu.sync_copy(hsc.at[ch, pl.ds(s * 640, 640)],
                        acc.at[pl.ds(s * 640, 640)])
        plsc.subcore_barrier()

        def step(j, _):
            pltpu.sync_copy(hsc.at[ch].at[srcv.at[j]], rows)
            for k in range(128 // L):
                iv = dstv[j, pl.ds(k * L, L)]
                pltpu.sync_copy(rows.at[pl.ds(k * L, L)], acc.at[iv],
                                add=True)
            return _

        lax.fori_loop(0, EPS // 128, step, None)
        plsc.subcore_barrier()
        pltpu.sync_copy(acc.at[pl.ds(s * 640, 640)],
                        out.at[ch, pl.ds(s * 640, 640)])
        plsc.subcore_barrier()


_agg_sc = functools.partial(
    pl.kernel, _agg_body,
    out_type=jax.ShapeDtypeStruct((4, NROWS, 128), _f32),
    mesh=_MESH,
    scratch_types=[
        pltpu.VMEM_SHARED((NROWS, 128), _f32),
        pltpu.VMEM((EPS // 128, 128), _i32),
        pltpu.VMEM((EPS // 128, 128), _i32),
        pltpu.VMEM((128, 128), _f32),
    ],
)()


def _pool_body(h2c, node3d, avg3d, idsall, z2, z1g, ones1, sums4, cntp,
               presf, acc, cacc, pacc, nv, av, idpv, rows, ones1v):
    c = lax.axis_index("c")
    s = lax.axis_index("s")
    pltpu.sync_copy(z2.at[pl.ds(s * 128, 128)], acc.at[pl.ds(s * 128, 128)])

    @pl.when(s == 0)
    def _():
        pltpu.sync_copy(z1g, cacc)
        pltpu.sync_copy(z1g, pacc)

    pltpu.sync_copy(ones1, ones1v)
    pltpu.sync_copy(node3d.at[s], nv)
    pltpu.sync_copy(avg3d.at[s], av)
    pltpu.sync_copy(idsall.at[s], idpv)
    plsc.subcore_barrier()

    @pl.when(c == 1)
    def _():
        for j in range(4):
            for k in range(128 // L):
                iv = idpv[j, pl.ds(k * L, L)]
                pltpu.sync_copy(ones1v, pacc.at[iv])

    for j2 in range(2):
        ch = c + 2 * j2

        def step(j, _):
            pltpu.sync_copy(h2c.at[ch].at[nv.at[j]], rows)
            for k in range(128 // L):
                iv = av[j, pl.ds(k * L, L)] + GPAD * j2
                pltpu.sync_copy(rows.at[pl.ds(k * L, L)], acc.at[iv],
                                add=True)
            return _

        lax.fori_loop(0, SGP // 128, step, None)

    def cstep(j, _):
        for k in range(128 // L):
            iv = av[j, pl.ds(k * L, L)]
            pltpu.sync_copy(ones1v, cacc.at[iv], add=True)
        return _

    half = SGP // 128 // 2
    lax.fori_loop(c * half, half + c * (SGP // 128 - half), cstep, None)

    plsc.subcore_barrier()
    for j2 in range(2):
        pltpu.sync_copy(acc.at[pl.ds(GPAD * j2 + s * 64, 64)],
                        sums4.at[c + 2 * j2, pl.ds(s * 64, 64)])

    @pl.when(s == 0)
    def _():
        pltpu.sync_copy(cacc, cntp.at[c])

    @pl.when(jnp.logical_and(c == 1, s == 0))
    def _():
        pltpu.sync_copy(pacc, presf)


_pool_sc = functools.partial(
    pl.kernel, _pool_body,
    out_type=(jax.ShapeDtypeStruct((4, GPAD, 128), _f32),
              jax.ShapeDtypeStruct((NC, GPAD), _f32),
              jax.ShapeDtypeStruct((GPAD,), _f32)),
    mesh=_MESH,
    scratch_types=[
        pltpu.VMEM_SHARED((2 * GPAD, 128), _f32),
        pltpu.VMEM_SHARED((GPAD,), _f32),
        pltpu.VMEM_SHARED((GPAD,), _f32),
        pltpu.VMEM((SGP // 128, 128), _i32),
        pltpu.VMEM((SGP // 128, 128), _i32),
        pltpu.VMEM((4, 128), _i32),
        pltpu.VMEM((128, 128), _f32),
        pltpu.VMEM((L,), _f32),
    ],
)()


def _pairs_body(from2d, to2d, rankshbm, sums4, cnthbm, ftrows, cfo, cto,
                fidv, tidv, rtmp, ctmp, cfv, rlist, rows):
    c = lax.axis_index("c")
    s = lax.axis_index("s")
    w = c * NS + s
    pltpu.sync_copy(from2d.at[w], fidv)
    pltpu.sync_copy(to2d.at[w], tidv)

    for b, (idv, c_out) in enumerate(((fidv, cfo), (tidv, cto))):
        for k in range(128 // L):
            ids16 = idv[0, pl.ds(k * L, L)]
            pltpu.sync_copy(rankshbm.at[ids16], rtmp)
            r16 = rtmp[...] - 1
            rlist[pl.ds(k * L, L)] = r16
            pltpu.sync_copy(cnthbm.at[r16], ctmp)
            cfv[pl.ds(k * L, L)] = jnp.maximum(ctmp[...], 1.0)
        pltpu.sync_copy(cfv, c_out.at[pl.ds(128 * w, 128)])
        for h in range(4):
            for k in range(128 // L):
                iv = rlist[pl.ds(k * L, L)]
                pltpu.sync_copy(sums4.at[h].at[iv],
                                rows.at[pl.ds(k * L, L)])
            pltpu.sync_copy(rows, ftrows.at[b, h, pl.ds(128 * w, 128)])


_pairs_sc = functools.partial(
    pl.kernel, _pairs_body,
    out_type=(jax.ShapeDtypeStruct((2, 4, N_PAIRS, 128), _f32),
              jax.ShapeDtypeStruct((N_PAIRS,), _f32),
              jax.ShapeDtypeStruct((N_PAIRS,), _f32)),
    mesh=_MESH,
    scratch_types=[
        pltpu.VMEM((1, 128), _i32),
        pltpu.VMEM((1, 128), _i32),
        pltpu.VMEM((L,), _i32),
        pltpu.VMEM((L,), _f32),
        pltpu.VMEM((128,), _f32),
        pltpu.VMEM((128,), _i32),
        pltpu.VMEM((128, 128), _f32),
    ],
)()


def _rank_body(p_ref, cp_ref, r_ref, c_ref):
    p = p_ref[...]
    row = lax.broadcasted_iota(_i32, (GPAD, GPAD), 0)
    col = lax.broadcasted_iota(_i32, (GPAD, GPAD), 1)
    m = jnp.where(col <= row, 1.0, 0.0)
    ranks = lax.dot_general(m, p, (((1,), (0,)), ((), ())),
                            precision=lax.Precision.HIGHEST,
                            preferred_element_type=_f32)
    r_ref[...] = jnp.round(ranks).astype(_i32)
    cp = cp_ref[...]
    c_ref[...] = (cp[0, :] + cp[1, :])[:, None]


def _rank_tc(presf, cntp):
    return pl.pallas_call(
        _rank_body,
        out_shape=[jax.ShapeDtypeStruct((GPAD, 1), _i32),
                   jax.ShapeDtypeStruct((GPAD, 1), _f32)],
    )(presf, cntp)


_BM = 1024


def _m1_body(x_ref, degp_ref, w_ref, hs_ref, dis_ref):
    dp = degp_ref[...]
    deg = 1.0 + dp[:, 0] + dp[:, 1]
    dis = lax.rsqrt(deg)
    h = jnp.dot(x_ref[...], w_ref[...], preferred_element_type=_f32)
    hs = h * dis[:, None]
    for c in range(4):
        hs_ref[c] = hs[:, 128 * c:128 * (c + 1)]
    dis_ref[...] = dis[:, None]


def _m1_tc(x, degp, W1):
    return pl.pallas_call(
        _m1_body,
        grid=(NROWS // _BM,),
        in_specs=[
            pl.BlockSpec((_BM, IN_CH), lambda i: (i, 0)),
            pl.BlockSpec((_BM, 2), lambda i: (i, 0)),
            pl.BlockSpec((IN_CH, EMB), lambda i: (0, 0)),
        ],
        out_specs=[
            pl.BlockSpec((4, _BM, 128), lambda i: (0, i, 0)),
            pl.BlockSpec((_BM, 1), lambda i: (i, 0)),
        ],
        out_shape=[
            jax.ShapeDtypeStruct((4, NROWS, 128), _f32),
            jax.ShapeDtypeStruct((NROWS, 1), _f32),
        ],
    )(x, degp, W1)


def _m2_body(agg_ref, dis_ref, b_ref, w_ref, hs_ref):
    a = jnp.concatenate([agg_ref[c] for c in range(4)], axis=-1)
    dis = dis_ref[...]
    h1 = jnp.maximum(a * dis + b_ref[...][0, :], 0.0)
    hs = jnp.dot(h1, w_ref[...], preferred_element_type=_f32) * dis
    for c in range(4):
        hs_ref[c] = hs[:, 128 * c:128 * (c + 1)]


def _m2_tc(aggc, dis, b1r, W2):
    return pl.pallas_call(
        _m2_body,
        grid=(NROWS // _BM,),
        in_specs=[
            pl.BlockSpec((4, _BM, 128), lambda i: (0, i, 0)),
            pl.BlockSpec((_BM, 1), lambda i: (i, 0)),
            pl.BlockSpec((1, EMB), lambda i: (0, 0)),
            pl.BlockSpec((EMB, EMB), lambda i: (0, 0)),
        ],
        out_specs=pl.BlockSpec((4, _BM, 128), lambda i: (0, i, 0)),
        out_shape=jax.ShapeDtypeStruct((4, NROWS, 128), _f32),
    )(aggc, dis, b1r, W2)


def _elem_body(agg_ref, dis_ref, b_ref, h2_ref):
    a = jnp.concatenate([agg_ref[c] for c in range(4)], axis=-1)
    h2 = jnp.maximum(a * dis_ref[...] + b_ref[...][0, :], 0.0)
    for c in range(4):
        h2_ref[c] = h2[:, 128 * c:128 * (c + 1)]


def _elem_tc(aggc, dis, b2r):
    return pl.pallas_call(
        _elem_body,
        grid=(NROWS // _BM,),
        in_specs=[
            pl.BlockSpec((4, _BM, 128), lambda i: (0, i, 0)),
            pl.BlockSpec((_BM, 1), lambda i: (i, 0)),
            pl.BlockSpec((1, EMB), lambda i: (0, 0)),
        ],
        out_specs=pl.BlockSpec((4, _BM, 128), lambda i: (0, i, 0)),
        out_shape=jax.ShapeDtypeStruct((4, NROWS, 128), _f32),
    )(aggc, dis, b2r)


def _dot_body(f_ref, t_ref, cf_ref, ct_ref, o_ref):
    f = f_ref[0]
    t = t_ref[0]
    sums = jnp.sum(f * t, axis=(0, 2))
    o_ref[...] = (sums / (cf_ref[...][:, 0] * ct_ref[...][:, 0]))[:, None]


def _dot_tc(frows, trows, cf2, ct2):
    bm = 1024
    return pl.pallas_call(
        _dot_body,
        grid=(N_PAIRS // bm,),
        in_specs=[
            pl.BlockSpec((1, 4, bm, 128), lambda i: (0, 0, i, 0)),
            pl.BlockSpec((1, 4, bm, 128), lambda i: (1, 0, i, 0)),
            pl.BlockSpec((bm, 1), lambda i: (i, 0)),
            pl.BlockSpec((bm, 1), lambda i: (i, 0)),
        ],
        out_specs=pl.BlockSpec((bm, 1), lambda i: (i, 0)),
        out_shape=jax.ShapeDtypeStruct((N_PAIRS, 1), _f32),
    )(frows, trows, cf2, ct2)


def kernel(x, drug_drug_batch, edge_cell_lines, sg_edge_index, sg_nodes,
           sg_avging_idx, W1, b1, W2, b2):
    del edge_cell_lines
    src = sg_edge_index[0].astype(_i32)
    dst = sg_edge_index[1].astype(_i32)

    pad_a = NS * EPS - N_EDGES
    spread_src = (jnp.arange(pad_a, dtype=_i32) * 37) % N_NODES
    spread_dst = N_NODES + (jnp.arange(pad_a, dtype=_i32) % (NROWS - N_NODES))
    src3d = jnp.concatenate([src, spread_src]).reshape(NS, EPS // 128, 128)
    dst3d = jnp.concatenate([dst, spread_dst]).reshape(NS, EPS // 128, 128)
    pad_d = NC * NS * EPD - N_EDGES
    spread_dd = N_NODES + (jnp.arange(pad_d, dtype=_i32) % (NROWS - N_NODES))
    dst4d = jnp.concatenate([dst, spread_dd]).reshape(NC, NS, EPD // 128, 128)
    pad_p = NS * SGP - N_SG
    spread_n = (jnp.arange(pad_p, dtype=_i32) * 53) % N_NODES
    spread_g = N_GRAPHS + (jnp.arange(pad_p, dtype=_i32) % (GPAD - N_GRAPHS))
    node3d = jnp.concatenate([sg_nodes.astype(_i32), spread_n]
                             ).reshape(NS, SGP // 128, 128)
    avg3d = jnp.concatenate([sg_avging_idx.astype(_i32), spread_g]
                            ).reshape(NS, SGP // 128, 128)
    ddb = drug_drug_batch.astype(_i32)
    from2d = ddb[0].reshape(NC * NS, 1, 128)
    to2d = ddb[1].reshape(NC * NS, 1, 128)
    idsall = ddb.reshape(NS, 4, 128)

    z1n = jnp.zeros((NROWS,), _f32)
    z2 = jnp.zeros((2 * GPAD, 128), _f32)
    z1g = jnp.zeros((GPAD,), _f32)
    ones1f = jnp.ones((L,), _f32)

    degp = _deg_sc(dst4d, z1n, ones1f)
    xp = jnp.concatenate([x, jnp.zeros((NROWS - N_NODES, IN_CH), _f32)])
    hs1c, dis = _m1_tc(xp, degp.T, W1)
    agg1c = _agg_sc(hs1c, src3d, dst3d)
    hs2c = _m2_tc(agg1c, dis, b1.reshape(1, EMB), W2)
    agg2c = _agg_sc(hs2c, src3d, dst3d)
    h2c = _elem_tc(agg2c, dis, b2.reshape(1, EMB))
    sums4, cntp, presf = _pool_sc(h2c, node3d, avg3d, idsall, z2, z1g,
                                  ones1f)
    ranks2, cnt2 = _rank_tc(presf.reshape(GPAD, 1), cntp)
    ftrows, cf, ct = _pairs_sc(from2d, to2d, ranks2.reshape(GPAD),
                               sums4, cnt2.reshape(GPAD))
    out = _dot_tc(ftrows, ftrows, cf.reshape(N_PAIRS, 1),
                  ct.reshape(N_PAIRS, 1))
    return out.reshape(N_PAIRS)
